# baseline (speedup 1.0000x reference)
import functools

import jax
import jax.numpy as jnp
from jax import lax
from jax.experimental import pallas as pl
from jax.experimental.pallas import tpu as pltpu
from jax.experimental.pallas import tpu_sc as plsc

NUM_CORES = 2
NUM_SUBCORES = 16
NUM_WORKERS = NUM_CORES * NUM_SUBCORES
LANES = 16


def _sc_segment_reduce(xs, p, batch, B):
  N, D = xs.shape
  assert D == 128
  rows_per = N // NUM_WORKERS
  assert rows_per * NUM_WORKERS == N
  CHUNK = 80
  assert rows_per % CHUNK == 0 and CHUNK % LANES == 0
  n_chunks = rows_per // CHUNK
  n_groups = CHUNK // LANES
  BP = ((B + LANES + 255) // 256) * 256
  zrows = BP // NUM_SUBCORES
  assert zrows % 8 == 0
  CBF = ((B + 1 + 127) // 128) * 128

  mesh = plsc.VectorSubcoreMesh(core_axis_name="c", subcore_axis_name="s")

  @functools.partial(
      pl.kernel,
      out_type=(
          jax.ShapeDtypeStruct((NUM_CORES, BP, D), jnp.float32),
          jax.ShapeDtypeStruct((NUM_WORKERS, CBF), jnp.float32),
      ),
      mesh=mesh,
      scratch_types=[
          pltpu.VMEM_SHARED((BP, D), jnp.float32),
          pltpu.VMEM((2, CHUNK, D), jnp.float32),
          pltpu.VMEM((2, CHUNK), jnp.int32),
          pltpu.VMEM((2, CHUNK), jnp.float32),
          pltpu.VMEM((16, D), jnp.float32),
          pltpu.VMEM((CBF,), jnp.float32),
          pltpu.VMEM((8 * LANES,), jnp.float32),
          pltpu.VMEM((LANES,), jnp.int32),
          pltpu.SemaphoreType.DMA((2,)),
      ],
  )
  def seg_kernel(xs_hbm, p_hbm, b_hbm, z_hbm, sum_hbm, cnt_hbm, shared_sum,
                 xs_buf, b_buf, p_buf, stage, cnt_grid, acc_ref, sidx_ref,
                 sems):
    cid = lax.axis_index("c")
    sid = lax.axis_index("s")
    wid = cid * NUM_SUBCORES + sid
    base = wid * rows_per
    lane = lax.iota(jnp.int32, LANES)
    zvec = jnp.zeros((LANES,), jnp.float32)
    one = jnp.int32(1)
    zero = jnp.int32(0)
    discard = jnp.int32(B)

    def lane_onehot(pos):
      return one - jnp.minimum(jnp.abs(lane - pos), one)

    def lane_ge(pos):
      return jnp.minimum(jnp.maximum(lane - pos + one, zero), one)

    onehot0 = lane_onehot(zero)
    onehot0_f = onehot0.astype(jnp.float32)
    zeros_i = lane * zero

    def vsplat(v, k):
      return v.at[zeros_i + k].get(mode="promise_in_bounds")

    def count_add(run_id, run_cnt):
      bpos = run_id & jnp.int32(-16)
      w = cnt_grid[pl.ds(bpos, LANES)]
      cnt_grid[pl.ds(bpos, LANES)] = (
          w + lane_onehot(run_id - bpos).astype(jnp.float32) * run_cnt)

    def count_bump(flush, run_id, run_cnt):
      @pl.when(flush)
      def _():
        count_add(run_id, run_cnt)

    def chunk_starts(c, par):
      row0 = base + c * CHUNK
      return (
          pltpu.make_async_copy(
              xs_hbm.at[pl.ds(row0, CHUNK)], xs_buf.at[par], sems.at[par]),
          pltpu.make_async_copy(
              b_hbm.at[pl.ds(row0, CHUNK)], b_buf.at[par], sems.at[par]),
          pltpu.make_async_copy(
              p_hbm.at[pl.ds(row0, CHUNK)], p_buf.at[par], sems.at[par]),
      )

    pltpu.sync_copy(z_hbm, shared_sum.at[pl.ds(sid * zrows, zrows)])
    plsc.subcore_barrier()

    def zero_cnt(r, _):
      cnt_grid[pl.ds(r * LANES, LANES)] = zvec
      return 0
    lax.fori_loop(0, CBF // LANES, zero_cnt, 0)
    for j in range(8):
      acc_ref[pl.ds(j * LANES, LANES)] = zvec
    sidx_ref[...] = jnp.full((LANES,), B, jnp.int32)
    for d in chunk_starts(0, 0) + chunk_starts(1, 1):
      d.start()

    def chunk_body(c, carry):
      par = lax.rem(c, 2)
      xbuf = xs_buf.at[par]
      for d in chunk_starts(c, par):
        d.wait()

      def group_body(g, carry):
        ids_v = b_buf.at[par][pl.ds(g * LANES, LANES)]
        pv_v = p_buf.at[par][pl.ds(g * LANES, LANES)]
        uniform = ids_v[LANES - 1] == carry[1]

        def fast_group(carry):
          cnt, prev_id, scount = carry
          prods = [
              tuple(
                  xbuf.at[g * LANES + k][pl.ds(j * LANES, LANES)]
                  * vsplat(pv_v, k)
                  for j in range(8))
              for k in range(LANES)
          ]
          while len(prods) > 1:
            prods = [
                tuple(a + b for a, b in zip(prods[i], prods[i + 1]))
                for i in range(0, len(prods), 2)
            ]
          srow = stage.at[scount]
          for j in range(8):
            a = acc_ref[pl.ds(j * LANES, LANES)] + prods[0][j]
            acc_ref[pl.ds(j * LANES, LANES)] = a
            srow[pl.ds(j * LANES, LANES)] = a
          return (cnt + float(LANES), prev_id, scount)

        def slow_group(carry):
          cnt, prev_id, scount = carry
          accs = tuple(
              acc_ref[pl.ds(j * LANES, LANES)] for j in range(8))
          sidx = sidx_ref[...]
          for k in range(LANES):
            bid = ids_v[k]
            pv = vsplat(pv_v, k)
            flush = jnp.logical_and(bid != prev_id, cnt != 0.0)
            fi = jnp.where(flush, one, zero)
            fif = fi.astype(jnp.float32)
            keep = 1.0 - fif
            count_bump(flush, prev_id, cnt)
            sel = lane_onehot(scount) * fi
            sidx = sidx * (one - sel) + prev_id * sel
            scount = scount + fi

            @pl.when(scount == 16)
            def _(sidx=sidx):
              pltpu.sync_copy(stage, shared_sum.at[sidx], add=True)

            scount = jnp.where(scount == 16, 0, scount)
            xrow = xbuf.at[g * LANES + k]
            accs = tuple(
                a * keep + xrow[pl.ds(j * LANES, LANES)] * pv
                for j, a in enumerate(accs))
            cnt = cnt * keep + 1.0
            srow = stage.at[scount]
            for j in range(8):
              srow[pl.ds(j * LANES, LANES)] = accs[j]
            prev_id = bid
          for j in range(8):
            acc_ref[pl.ds(j * LANES, LANES)] = accs[j]
          sidx_ref[...] = sidx
          return (cnt, prev_id, scount)

        return lax.cond(uniform, fast_group, slow_group, carry)

      carry = lax.fori_loop(0, n_groups, group_body, carry)

      @pl.when(c + 2 < n_chunks)
      def _():
        for d in chunk_starts(c + 2, par):
          d.start()

      return carry

    init = (0.0, jnp.int32(-1), jnp.int32(0))
    cnt, prev_id, scount = lax.fori_loop(0, n_chunks, chunk_body, init)

    count_add(prev_id, cnt)
    sel = lane_onehot(scount)
    sidx = sidx_ref[...] * (one - sel) + prev_id * sel
    scount = scount + 1
    ge = lane_ge(scount)
    sidx = sidx * (one - ge) + discard * ge
    pltpu.sync_copy(stage, shared_sum.at[sidx], add=True)

    plsc.subcore_barrier()
    pltpu.sync_copy(shared_sum.at[pl.ds(sid * zrows, zrows)],
                    sum_hbm.at[cid, pl.ds(sid * zrows, zrows)])
    pltpu.sync_copy(cnt_grid, cnt_hbm.at[wid])

  zeros = jnp.zeros((zrows, D), jnp.float32)
  return seg_kernel(xs, p, batch, zeros), BP, CBF


def _combine(x_root, sums, cnt, B):
  D = x_root.shape[1]
  RB = 400
  assert B % RB == 0

  def body(xr_ref, sum_ref, cnt_ref, o_ref):
    s = sum_ref[0] + sum_ref[1]
    c = jnp.maximum(jnp.sum(cnt_ref[...], axis=0), 1.0)
    o_ref[:, :D] = xr_ref[...]
    o_ref[:, D:] = s / c

  return pl.pallas_call(
      body,
      grid=(B // RB,),
      in_specs=[
          pl.BlockSpec((RB, D), lambda i: (i, 0)),
          pl.BlockSpec((NUM_CORES, RB, D), lambda i: (0, i, 0)),
          pl.BlockSpec((NUM_WORKERS, RB, 1), lambda i: (0, i, 0)),
      ],
      out_specs=pl.BlockSpec((RB, 2 * D), lambda i: (i, 0)),
      out_shape=jax.ShapeDtypeStruct((B, 2 * D), jnp.float32),
  )(x_root, sums, cnt)


def kernel(x_root, xs, p, batch):
  B = x_root.shape[0]
  batch = batch.astype(jnp.int32)
  (sums, cnts), BP, CBF = _sc_segment_reduce(xs, p, batch, B)
  cnt = cnts[:, :B].reshape(NUM_WORKERS, B, 1)
  return _combine(x_root, sums, cnt, B)
